# baseline (speedup 1.0000x reference)
import functools

import jax
import jax.numpy as jnp
from jax.experimental import pallas as pl
from jax.experimental.pallas import tpu as pltpu

_DD = 4096
_BB = 4096


def _body(nd, dd, d_total, s_ref, e_ref, c_ref, o_ref, q_ref, p_ref, w_ref):
    j = pl.program_id(0)

    def _dots(e, c):
        e16 = e.astype(jnp.bfloat16)
        c16 = c.astype(jnp.bfloat16)
        q = jax.lax.dot_general(e16, e16, (((0,), (0,)), ((), ())),
                                preferred_element_type=jnp.float32)
        pt = jax.lax.dot_general(c16, e16, (((1,), (0,)), ((), ())),
                                 preferred_element_type=jnp.float32)
        w = jnp.sum(c * c, axis=1, keepdims=True)
        return q, pt, w

    @pl.when(j == 0)
    def _s1_first():
        q, pt, w = _dots(e_ref[...], c_ref[...])
        q_ref[...] = q
        p_ref[...] = pt
        w_ref[...] = w

    @pl.when((j > 0) & (j < nd - 1))
    def _s1_interior():
        q, pt, w = _dots(e_ref[...], c_ref[...])
        q_ref[...] += q
        p_ref[...] += pt
        w_ref[...] += w

    @pl.when(j == nd - 1)
    def _s1_edge():
        lim = d_total - (nd - 1) * dd
        e = e_ref[...]
        c = c_ref[...]
        e = jnp.where(jax.lax.broadcasted_iota(jnp.int32, e.shape, 0) < lim,
                      e, 0.0)
        c = jnp.where(jax.lax.broadcasted_iota(jnp.int32, c.shape, 1) < lim,
                      c, 0.0)
        q, pt, w = _dots(e, c)
        q_ref[...] += q
        p_ref[...] += pt
        w_ref[...] += w

    @pl.when(j >= nd)
    def _stage2():
        s = s_ref[...]
        s16 = s.astype(jnp.bfloat16)
        q16 = q_ref[...].astype(jnp.bfloat16)
        pt16 = p_ref[...].astype(jnp.bfloat16)
        sq = jax.lax.dot_general(s16, q16, (((1,), (0,)), ((), ())),
                                 preferred_element_type=jnp.float32)
        ensq = jnp.sum(sq * s, axis=1, keepdims=True)
        raw = jax.lax.dot_general(s16, pt16, (((1,), (1,)), ((), ())),
                                  preferred_element_type=jnp.float32)
        en = jnp.maximum(jnp.sqrt(jnp.maximum(ensq, 0.0)), 1e-12)
        wn = jnp.maximum(jnp.sqrt(w_ref[...]), 1e-12)
        wn_row = wn.reshape((1, wn.shape[0]))
        o_ref[...] = raw / (en * wn_row)


def kernel(samples, enc_weight, cent_weight):
    B, F = samples.shape
    D = enc_weight.shape[0]
    C = cent_weight.shape[0]
    nd = -(-D // _DD)
    nb = B // _BB

    return pl.pallas_call(
        functools.partial(_body, nd, _DD, D),
        grid=(nd + nb,),
        in_specs=[
            pl.BlockSpec((_BB, F), lambda j: (jnp.maximum(j - nd, 0), 0)),
            pl.BlockSpec((_DD, F), lambda j: (jnp.minimum(j, nd - 1), 0)),
            pl.BlockSpec((C, _DD), lambda j: (0, jnp.minimum(j, nd - 1))),
        ],
        out_specs=pl.BlockSpec((_BB, C), lambda j: (jnp.maximum(j - nd, 0), 0)),
        out_shape=jax.ShapeDtypeStruct((B, C), jnp.float32),
        scratch_shapes=[
            pltpu.VMEM((F, F), jnp.float32),
            pltpu.VMEM((C, F), jnp.float32),
            pltpu.VMEM((C, 1), jnp.float32),
        ],
        compiler_params=pltpu.CompilerParams(
            dimension_semantics=("arbitrary",)),
    )(samples, enc_weight, cent_weight)


# speedup vs baseline: 1.0938x; 1.0938x over previous
import functools

import jax
import jax.numpy as jnp
from jax.experimental import pallas as pl
from jax.experimental.pallas import tpu as pltpu

_DD = 2048
_BB = 2048


def _body(nd, dd, d_total, s_ref, e_ref, c_ref, o_ref, q_ref, p_ref, w_ref):
    j = pl.program_id(0)

    def _dots(e, c):
        e16 = e.astype(jnp.bfloat16)
        c16 = c.astype(jnp.bfloat16)
        q = jax.lax.dot_general(e16, e16, (((0,), (0,)), ((), ())),
                                preferred_element_type=jnp.float32)
        pt = jax.lax.dot_general(c16, e16, (((1,), (0,)), ((), ())),
                                 preferred_element_type=jnp.float32)
        w = jnp.sum(c * c, axis=1, keepdims=True)
        return q, pt, w

    @pl.when(j == 0)
    def _s1_first():
        q, pt, w = _dots(e_ref[...], c_ref[...])
        q_ref[...] = q
        p_ref[...] = pt
        w_ref[...] = w

    @pl.when((j > 0) & (j < nd - 1))
    def _s1_interior():
        q, pt, w = _dots(e_ref[...], c_ref[...])
        q_ref[...] += q
        p_ref[...] += pt
        w_ref[...] += w

    @pl.when(j == nd - 1)
    def _s1_edge():
        lim = d_total - (nd - 1) * dd
        e = e_ref[...]
        c = c_ref[...]
        e = jnp.where(jax.lax.broadcasted_iota(jnp.int32, e.shape, 0) < lim,
                      e, 0.0)
        c = jnp.where(jax.lax.broadcasted_iota(jnp.int32, c.shape, 1) < lim,
                      c, 0.0)
        q, pt, w = _dots(e, c)
        q_ref[...] += q
        p_ref[...] += pt
        w_ref[...] += w

    @pl.when(j >= nd)
    def _stage2():
        s = s_ref[...]
        s16 = s.astype(jnp.bfloat16)
        q16 = q_ref[...].astype(jnp.bfloat16)
        pt16 = p_ref[...].astype(jnp.bfloat16)
        sq = jax.lax.dot_general(s16, q16, (((1,), (0,)), ((), ())),
                                 preferred_element_type=jnp.float32)
        ensq = jnp.sum(sq * s, axis=1, keepdims=True)
        raw = jax.lax.dot_general(s16, pt16, (((1,), (1,)), ((), ())),
                                  preferred_element_type=jnp.float32)
        en = jnp.maximum(jnp.sqrt(jnp.maximum(ensq, 0.0)), 1e-12)
        wn = jnp.maximum(jnp.sqrt(w_ref[...]), 1e-12)
        wn_row = wn.reshape((1, wn.shape[0]))
        o_ref[...] = raw / (en * wn_row)


def kernel(samples, enc_weight, cent_weight):
    B, F = samples.shape
    D = enc_weight.shape[0]
    C = cent_weight.shape[0]
    nd = -(-D // _DD)
    nb = B // _BB

    return pl.pallas_call(
        functools.partial(_body, nd, _DD, D),
        grid=(nd + nb,),
        in_specs=[
            pl.BlockSpec((_BB, F), lambda j: (jnp.maximum(j - nd, 0), 0)),
            pl.BlockSpec((_DD, F), lambda j: (jnp.minimum(j, nd - 1), 0)),
            pl.BlockSpec((C, _DD), lambda j: (0, jnp.minimum(j, nd - 1))),
        ],
        out_specs=pl.BlockSpec((_BB, C), lambda j: (jnp.maximum(j - nd, 0), 0)),
        out_shape=jax.ShapeDtypeStruct((B, C), jnp.float32),
        scratch_shapes=[
            pltpu.VMEM((F, F), jnp.float32),
            pltpu.VMEM((C, F), jnp.float32),
            pltpu.VMEM((C, 1), jnp.float32),
        ],
        compiler_params=pltpu.CompilerParams(
            dimension_semantics=("arbitrary",)),
    )(samples, enc_weight, cent_weight)
